# Initial kernel scaffold; baseline (speedup 1.0000x reference)
import jax
import jax.numpy as jnp
from jax.experimental import pallas as pl


def kernel(x, adj, W, b):
    raise NotImplementedError("write your pallas kernel here")


# speedup vs baseline: 21.3852x; 21.3852x over previous
import functools

import jax
import jax.numpy as jnp
from jax import lax
from jax.experimental import pallas as pl
from jax.experimental.pallas import tpu as pltpu
from jax.experimental.pallas import tpu_sc as plsc

N = 10000
N_PAD = 10240
E = 160000
E_PAD = 163840
F_IN = 256
F_OUT = 128
C = 128
ROWS_T = N_PAD // 16
CHUNKS = E_PAD // C
CPW = CHUNKS // 32

_MESH = plsc.VectorSubcoreMesh(core_axis_name="c", subcore_axis_name="s")


@functools.partial(
    pl.kernel,
    out_type=jax.ShapeDtypeStruct((2, N_PAD), jnp.float32),
    mesh=_MESH,
    scratch_types=[
        pltpu.VMEM((CPW, C), jnp.int32),
        pltpu.VMEM((C,), jnp.float32),
        pltpu.VMEM((ROWS_T,), jnp.float32),
        pltpu.VMEM_SHARED((N_PAD,), jnp.float32),
    ],
)
def _deg_kernel(dst2d, deg_pair, dsts, ones, zer, deg_sh):
    c = lax.axis_index("c")
    s = lax.axis_index("s")
    wid = c * 16 + s
    row0 = s * ROWS_T
    for i in range(C // 16):
        ones[pl.ds(i * 16, 16)] = jnp.ones((16,), jnp.float32)
    for i in range(ROWS_T // 16):
        zer[pl.ds(i * 16, 16)] = jnp.zeros((16,), jnp.float32)
    pltpu.sync_copy(zer, deg_sh.at[pl.ds(row0, ROWS_T)])
    pltpu.sync_copy(dst2d.at[pl.ds(wid * CPW, CPW)], dsts)
    plsc.subcore_barrier()

    @pl.loop(0, CPW)
    def _(j):
        pltpu.sync_copy(ones, deg_sh.at[dsts.at[j]], add=True)

    plsc.subcore_barrier()
    pltpu.sync_copy(deg_sh.at[pl.ds(row0, ROWS_T)], deg_pair.at[c, pl.ds(row0, ROWS_T)])


@functools.partial(
    pl.kernel,
    out_type=jax.ShapeDtypeStruct((2, N_PAD, F_OUT), jnp.float32),
    mesh=_MESH,
    scratch_types=[
        pltpu.VMEM((CPW, C), jnp.int32),
        pltpu.VMEM((CPW, C), jnp.int32),
        pltpu.VMEM((C, F_OUT), jnp.float32),
        pltpu.VMEM((16, F_OUT), jnp.float32),
        pltpu.VMEM_SHARED((N_PAD, F_OUT), jnp.float32),
        pltpu.SemaphoreType.DMA,
    ],
)
def _prop_kernel(g, src2d, dst2d, t_pair, srcs, dsts, rows, zbuf, acc, sem):
    c = lax.axis_index("c")
    s = lax.axis_index("s")
    wid = c * 16 + s
    row0 = s * ROWS_T
    for i in range(16 * F_OUT // 16):
        zbuf[pl.ds((i // 8) * 1, 1), pl.ds((i % 8) * 16, 16)] = jnp.zeros(
            (1, 16), jnp.float32
        )

    @pl.loop(0, ROWS_T // 16)
    def _(j):
        pltpu.sync_copy(zbuf, acc.at[pl.ds(row0 + j * 16, 16)])

    pltpu.sync_copy(src2d.at[pl.ds(wid * CPW, CPW)], srcs)
    pltpu.sync_copy(dst2d.at[pl.ds(wid * CPW, CPW)], dsts)
    plsc.subcore_barrier()

    @pl.loop(0, CPW)
    def _(j):
        pltpu.async_copy(g.at[srcs.at[j]], rows, sem).wait()
        pltpu.sync_copy(rows, acc.at[dsts.at[j]], add=True)

    plsc.subcore_barrier()
    pltpu.sync_copy(acc.at[pl.ds(row0, ROWS_T)], t_pair.at[c, pl.ds(row0, ROWS_T)])


def _mm_body(x_ref, w_ref, degp_ref, s0_ref, dinv_ref, dinv2_ref):
    deg = degp_ref[0] + degp_ref[1] + 1.0
    dinv = lax.rsqrt(deg)
    z = jax.lax.dot_general(
        x_ref[...], w_ref[...], (((1,), (1,)), ((), ())),
        preferred_element_type=jnp.float32,
    )
    s0_ref[...] = z * dinv[:, None]
    dinv_ref[...] = dinv
    dinv2_ref[...] = 1.0 / deg


def _mm_call(x_pad, w, deg_pair):
    bm = 512
    return pl.pallas_call(
        _mm_body,
        grid=(N_PAD // bm,),
        in_specs=[
            pl.BlockSpec((bm, F_IN), lambda i: (i, 0)),
            pl.BlockSpec((F_OUT, F_IN), lambda i: (0, 0)),
            pl.BlockSpec((2, bm), lambda i: (0, i)),
        ],
        out_specs=[
            pl.BlockSpec((bm, F_OUT), lambda i: (i, 0)),
            pl.BlockSpec((bm,), lambda i: (i,)),
            pl.BlockSpec((bm,), lambda i: (i,)),
        ],
        out_shape=[
            jax.ShapeDtypeStruct((N_PAD, F_OUT), jnp.float32),
            jax.ShapeDtypeStruct((N_PAD,), jnp.float32),
            jax.ShapeDtypeStruct((N_PAD,), jnp.float32),
        ],
    )(x_pad, w, deg_pair)


def _mid_body(tp_ref, g_ref, dinv2_ref, s_ref):
    t = tp_ref[0] + tp_ref[1] + g_ref[...]
    s_ref[...] = t * dinv2_ref[...][:, None]


def _mid_call(t_pair, g, dinv2):
    bm = 512
    return pl.pallas_call(
        _mid_body,
        grid=(N_PAD // bm,),
        in_specs=[
            pl.BlockSpec((2, bm, F_OUT), lambda i: (0, i, 0)),
            pl.BlockSpec((bm, F_OUT), lambda i: (i, 0)),
            pl.BlockSpec((bm,), lambda i: (i,)),
        ],
        out_specs=pl.BlockSpec((bm, F_OUT), lambda i: (i, 0)),
        out_shape=jax.ShapeDtypeStruct((N_PAD, F_OUT), jnp.float32),
    )(t_pair, g, dinv2)


def _fin_body(tp_ref, g_ref, dinv_ref, b_ref, o_ref):
    t = tp_ref[0] + tp_ref[1] + g_ref[...]
    o_ref[...] = t * dinv_ref[...][:, None] + b_ref[...][None, :]


def _fin_call(t_pair, g, dinv, b):
    bm = 512
    return pl.pallas_call(
        _fin_body,
        grid=(N_PAD // bm,),
        in_specs=[
            pl.BlockSpec((2, bm, F_OUT), lambda i: (0, i, 0)),
            pl.BlockSpec((bm, F_OUT), lambda i: (i, 0)),
            pl.BlockSpec((bm,), lambda i: (i,)),
            pl.BlockSpec((F_OUT,), lambda i: (0,)),
        ],
        out_specs=pl.BlockSpec((bm, F_OUT), lambda i: (i, 0)),
        out_shape=jax.ShapeDtypeStruct((N_PAD, F_OUT), jnp.float32),
    )(t_pair, g, dinv, b)


def kernel(x, adj, W, b):
    src = adj[0].astype(jnp.int32)
    dst = adj[1].astype(jnp.int32)
    padidx = N + (jnp.arange(E_PAD - E, dtype=jnp.int32) % (N_PAD - N))
    src2d = jnp.concatenate([src, padidx]).reshape(CHUNKS, C)
    dst2d = jnp.concatenate([dst, padidx]).reshape(CHUNKS, C)
    x_pad = jnp.concatenate(
        [x, jnp.zeros((N_PAD - N, F_IN), jnp.float32)], axis=0
    )

    deg_pair = _deg_kernel(dst2d)
    s0, dinv, dinv2 = _mm_call(x_pad, W, deg_pair)
    t0 = _prop_kernel(s0, src2d, dst2d)
    s1 = _mid_call(t0, s0, dinv2)
    t1 = _prop_kernel(s1, src2d, dst2d)
    out = _fin_call(t1, s1, dinv, b)
    return out[:N]


# speedup vs baseline: 28.3229x; 1.3244x over previous
import functools

import jax
import jax.numpy as jnp
from jax import lax
from jax.experimental import pallas as pl
from jax.experimental.pallas import tpu as pltpu
from jax.experimental.pallas import tpu_sc as plsc

N = 10000
N_PAD = 10240
E = 160000
E_PAD = 163840
F_IN = 256
F_OUT = 128
C = 128
ROWS_T = N_PAD // 16
CHUNKS = E_PAD // C
CPW = CHUNKS // 32

_MESH = plsc.VectorSubcoreMesh(core_axis_name="c", subcore_axis_name="s")


@functools.partial(
    pl.kernel,
    out_type=jax.ShapeDtypeStruct((2, N_PAD), jnp.float32),
    mesh=_MESH,
    scratch_types=[
        pltpu.VMEM((CPW, C), jnp.int32),
        pltpu.VMEM((C,), jnp.float32),
        pltpu.VMEM((ROWS_T,), jnp.float32),
        pltpu.VMEM_SHARED((N_PAD,), jnp.float32),
    ],
)
def _deg_kernel(dst2d, deg_pair, dsts, ones, zer, deg_sh):
    c = lax.axis_index("c")
    s = lax.axis_index("s")
    wid = c * 16 + s
    row0 = s * ROWS_T
    for i in range(C // 16):
        ones[pl.ds(i * 16, 16)] = jnp.ones((16,), jnp.float32)
    for i in range(ROWS_T // 16):
        zer[pl.ds(i * 16, 16)] = jnp.zeros((16,), jnp.float32)
    pltpu.sync_copy(zer, deg_sh.at[pl.ds(row0, ROWS_T)])
    pltpu.sync_copy(dst2d.at[pl.ds(wid * CPW, CPW)], dsts)
    plsc.subcore_barrier()

    @pl.loop(0, CPW)
    def _(j):
        pltpu.sync_copy(ones, deg_sh.at[dsts.at[j]], add=True)

    plsc.subcore_barrier()
    pltpu.sync_copy(deg_sh.at[pl.ds(row0, ROWS_T)], deg_pair.at[c, pl.ds(row0, ROWS_T)])


@functools.partial(
    pl.kernel,
    out_type=jax.ShapeDtypeStruct((2, N_PAD, F_OUT), jnp.float32),
    mesh=_MESH,
    scratch_types=[
        pltpu.VMEM((CPW, C), jnp.int32),
        pltpu.VMEM((CPW, C), jnp.int32),
        pltpu.VMEM((C, F_OUT), jnp.float32),
        pltpu.VMEM((C, F_OUT), jnp.float32),
        pltpu.VMEM((16, F_OUT), jnp.float32),
        pltpu.VMEM_SHARED((N_PAD, F_OUT), jnp.float32),
        pltpu.SemaphoreType.DMA,
        pltpu.SemaphoreType.DMA,
    ],
)
def _prop_kernel(
    g, src2d, dst2d, t_pair, srcs, dsts, rows0, rows1, zbuf, acc, sem0, sem1
):
    c = lax.axis_index("c")
    s = lax.axis_index("s")
    wid = c * 16 + s
    row0 = s * ROWS_T

    @pl.when(c == 0)
    def _():
        pltpu.sync_copy(g.at[pl.ds(row0, ROWS_T)], acc.at[pl.ds(row0, ROWS_T)])

    @pl.when(c == 1)
    def _():
        for i in range(16 * F_OUT // 16):
            zbuf[pl.ds(i // 8, 1), pl.ds((i % 8) * 16, 16)] = jnp.zeros(
                (1, 16), jnp.float32
            )

        @pl.loop(0, ROWS_T // 16)
        def _(j):
            pltpu.sync_copy(zbuf, acc.at[pl.ds(row0 + j * 16, 16)])

    pltpu.sync_copy(src2d.at[pl.ds(wid * CPW, CPW)], srcs)
    pltpu.sync_copy(dst2d.at[pl.ds(wid * CPW, CPW)], dsts)
    plsc.subcore_barrier()

    pltpu.async_copy(g.at[srcs.at[0]], rows0, sem0)

    @pl.loop(0, CPW, step=2)
    def _(j):
        pltpu.async_copy(g.at[srcs.at[j + 1]], rows1, sem1)
        pltpu.make_async_copy(g.at[srcs.at[j]], rows0, sem0).wait()
        pltpu.sync_copy(rows0, acc.at[dsts.at[j]], add=True)

        @pl.when(j + 2 < CPW)
        def _():
            pltpu.async_copy(g.at[srcs.at[j + 2]], rows0, sem0)

        pltpu.make_async_copy(g.at[srcs.at[j + 1]], rows1, sem1).wait()
        pltpu.sync_copy(rows1, acc.at[dsts.at[j + 1]], add=True)

    plsc.subcore_barrier()
    pltpu.sync_copy(acc.at[pl.ds(row0, ROWS_T)], t_pair.at[c, pl.ds(row0, ROWS_T)])


def _mm_body(x_ref, w_ref, degp_ref, s0_ref, dinv_ref, dinv2_ref):
    deg = degp_ref[0] + degp_ref[1] + 1.0
    dinv = lax.rsqrt(deg)
    z = jax.lax.dot_general(
        x_ref[...], w_ref[...], (((1,), (1,)), ((), ())),
        preferred_element_type=jnp.float32,
    )
    s0_ref[...] = z * dinv[:, None]
    dinv_ref[...] = dinv
    dinv2_ref[...] = 1.0 / deg


def _mm_call(x_pad, w, deg_pair):
    bm = 512
    return pl.pallas_call(
        _mm_body,
        grid=(N_PAD // bm,),
        in_specs=[
            pl.BlockSpec((bm, F_IN), lambda i: (i, 0)),
            pl.BlockSpec((F_OUT, F_IN), lambda i: (0, 0)),
            pl.BlockSpec((2, bm), lambda i: (0, i)),
        ],
        out_specs=[
            pl.BlockSpec((bm, F_OUT), lambda i: (i, 0)),
            pl.BlockSpec((bm,), lambda i: (i,)),
            pl.BlockSpec((bm,), lambda i: (i,)),
        ],
        out_shape=[
            jax.ShapeDtypeStruct((N_PAD, F_OUT), jnp.float32),
            jax.ShapeDtypeStruct((N_PAD,), jnp.float32),
            jax.ShapeDtypeStruct((N_PAD,), jnp.float32),
        ],
    )(x_pad, w, deg_pair)


def _mid_body(tp_ref, dinv2_ref, s_ref):
    t = tp_ref[0] + tp_ref[1]
    s_ref[...] = t * dinv2_ref[...][:, None]


def _mid_call(t_pair, dinv2):
    bm = 512
    return pl.pallas_call(
        _mid_body,
        grid=(N_PAD // bm,),
        in_specs=[
            pl.BlockSpec((2, bm, F_OUT), lambda i: (0, i, 0)),
            pl.BlockSpec((bm,), lambda i: (i,)),
        ],
        out_specs=pl.BlockSpec((bm, F_OUT), lambda i: (i, 0)),
        out_shape=jax.ShapeDtypeStruct((N_PAD, F_OUT), jnp.float32),
    )(t_pair, dinv2)


def _fin_body(tp_ref, dinv_ref, b_ref, o_ref):
    t = tp_ref[0] + tp_ref[1]
    o_ref[...] = t * dinv_ref[...][:, None] + b_ref[...][None, :]


def _fin_call(t_pair, dinv, b):
    bm = 512
    return pl.pallas_call(
        _fin_body,
        grid=(N_PAD // bm,),
        in_specs=[
            pl.BlockSpec((2, bm, F_OUT), lambda i: (0, i, 0)),
            pl.BlockSpec((bm,), lambda i: (i,)),
            pl.BlockSpec((F_OUT,), lambda i: (0,)),
        ],
        out_specs=pl.BlockSpec((bm, F_OUT), lambda i: (i, 0)),
        out_shape=jax.ShapeDtypeStruct((N_PAD, F_OUT), jnp.float32),
    )(t_pair, dinv, b)


def kernel(x, adj, W, b):
    src = adj[0].astype(jnp.int32)
    dst = adj[1].astype(jnp.int32)
    padidx = N + (jnp.arange(E_PAD - E, dtype=jnp.int32) % (N_PAD - N))
    src2d = jnp.concatenate([src, padidx]).reshape(CHUNKS, C)
    dst2d = jnp.concatenate([dst, padidx]).reshape(CHUNKS, C)
    x_pad = jnp.concatenate(
        [x, jnp.zeros((N_PAD - N, F_IN), jnp.float32)], axis=0
    )

    deg_pair = _deg_kernel(dst2d)
    s0, dinv, dinv2 = _mm_call(x_pad, W, deg_pair)
    t0 = _prop_kernel(s0, src2d, dst2d)
    s1 = _mid_call(t0, dinv2)
    t1 = _prop_kernel(s1, src2d, dst2d)
    out = _fin_call(t1, dinv, b)
    return out[:N]


# speedup vs baseline: 29.9464x; 1.0573x over previous
import functools

import jax
import jax.numpy as jnp
from jax import lax
from jax.experimental import pallas as pl
from jax.experimental.pallas import tpu as pltpu
from jax.experimental.pallas import tpu_sc as plsc

N = 10000
N_PAD = 10240
E = 160000
E_PAD = 163840
F_IN = 256
F_OUT = 128
C = 128
CG = 64
ROWS_T = N_PAD // 16
CHUNKS = E_PAD // C
CPW = CHUNKS // 32
CHUNKS_G = E_PAD // CG
CPWG = CHUNKS_G // 32

_MESH = plsc.VectorSubcoreMesh(core_axis_name="c", subcore_axis_name="s")


@functools.partial(
    pl.kernel,
    out_type=jax.ShapeDtypeStruct((2, N_PAD), jnp.float32),
    mesh=_MESH,
    scratch_types=[
        pltpu.VMEM((CPW, C), jnp.int32),
        pltpu.VMEM((C,), jnp.float32),
        pltpu.VMEM((ROWS_T,), jnp.float32),
        pltpu.VMEM_SHARED((N_PAD,), jnp.float32),
    ],
)
def _deg_kernel(dst2d, deg_pair, dsts, ones, zer, deg_sh):
    c = lax.axis_index("c")
    s = lax.axis_index("s")
    wid = c * 16 + s
    row0 = s * ROWS_T
    for i in range(C // 16):
        ones[pl.ds(i * 16, 16)] = jnp.ones((16,), jnp.float32)
    for i in range(ROWS_T // 16):
        zer[pl.ds(i * 16, 16)] = jnp.zeros((16,), jnp.float32)
    pltpu.sync_copy(zer, deg_sh.at[pl.ds(row0, ROWS_T)])
    pltpu.sync_copy(dst2d.at[pl.ds(wid * CPW, CPW)], dsts)
    plsc.subcore_barrier()

    @pl.loop(0, CPW)
    def _(j):
        pltpu.sync_copy(ones, deg_sh.at[dsts.at[j]], add=True)

    plsc.subcore_barrier()
    pltpu.sync_copy(deg_sh.at[pl.ds(row0, ROWS_T)], deg_pair.at[c, pl.ds(row0, ROWS_T)])


@functools.partial(
    pl.kernel,
    out_type=jax.ShapeDtypeStruct((2, N_PAD, F_OUT), jnp.float32),
    mesh=_MESH,
    scratch_types=[
        pltpu.VMEM((8, CG), jnp.int32),
        pltpu.VMEM((8, CG), jnp.int32),
        pltpu.VMEM((4, CG, F_OUT), jnp.float32),
        pltpu.VMEM((16, F_OUT), jnp.float32),
        pltpu.VMEM_SHARED((N_PAD, F_OUT), jnp.float32),
        [pltpu.SemaphoreType.DMA] * 4,
        [pltpu.SemaphoreType.DMA] * 8,
        [pltpu.SemaphoreType.DMA] * 8,
    ],
)
def _prop_kernel(
    g, src2d, dst2d, t_pair, sidx, didx, rows, zbuf, acc, gsem, ssem, dsem
):
    c = lax.axis_index("c")
    s = lax.axis_index("s")
    wid = c * 16 + s
    row0 = s * ROWS_T
    ch0 = wid * CPWG

    @pl.when(c == 0)
    def _():
        pltpu.sync_copy(g.at[pl.ds(row0, ROWS_T)], acc.at[pl.ds(row0, ROWS_T)])

    @pl.when(c == 1)
    def _():
        for i in range(16 * F_OUT // 16):
            zbuf[pl.ds(i // 8, 1), pl.ds((i % 8) * 16, 16)] = jnp.zeros(
                (1, 16), jnp.float32
            )

        @pl.loop(0, ROWS_T // 16)
        def _(j):
            pltpu.sync_copy(zbuf, acc.at[pl.ds(row0 + j * 16, 16)])

    plsc.subcore_barrier()

    for k in range(8):
        pltpu.async_copy(src2d.at[pl.ds(ch0 + k, 1)], sidx.at[pl.ds(k, 1)], ssem[k])
        pltpu.async_copy(dst2d.at[pl.ds(ch0 + k, 1)], didx.at[pl.ds(k, 1)], dsem[k])
    for k in range(4):
        pltpu.make_async_copy(
            src2d.at[pl.ds(ch0 + k, 1)], sidx.at[pl.ds(k, 1)], ssem[k]
        ).wait()
        pltpu.async_copy(g.at[sidx.at[k]], rows.at[k], gsem[k])

    @pl.loop(0, CPWG, step=8)
    def _(j):
        for b in range(8):
            rb = b % 4
            pltpu.make_async_copy(g.at[sidx.at[b]], rows.at[rb], gsem[rb]).wait()
            pltpu.make_async_copy(
                dst2d.at[pl.ds(ch0, 1)], didx.at[pl.ds(b, 1)], dsem[b]
            ).wait()
            pltpu.sync_copy(rows.at[rb], acc.at[didx.at[b]], add=True)

            @pl.when(j + b + 8 < CPWG)
            def _():
                pltpu.async_copy(
                    src2d.at[pl.ds(ch0 + j + b + 8, 1)],
                    sidx.at[pl.ds(b, 1)],
                    ssem[b],
                )
                pltpu.async_copy(
                    dst2d.at[pl.ds(ch0 + j + b + 8, 1)],
                    didx.at[pl.ds(b, 1)],
                    dsem[b],
                )

            @pl.when(j + b + 4 < CPWG)
            def _():
                b4 = (b + 4) % 8
                pltpu.make_async_copy(
                    src2d.at[pl.ds(ch0, 1)], sidx.at[pl.ds(b4, 1)], ssem[b4]
                ).wait()
                pltpu.async_copy(g.at[sidx.at[b4]], rows.at[rb], gsem[rb])

    plsc.subcore_barrier()
    pltpu.sync_copy(acc.at[pl.ds(row0, ROWS_T)], t_pair.at[c, pl.ds(row0, ROWS_T)])


def _mm_body(x_ref, w_ref, degp_ref, s0_ref, dinv_ref, dinv2_ref):
    deg = degp_ref[0] + degp_ref[1] + 1.0
    dinv = lax.rsqrt(deg)
    z = jax.lax.dot_general(
        x_ref[...], w_ref[...], (((1,), (1,)), ((), ())),
        preferred_element_type=jnp.float32,
    )
    s0_ref[...] = z * dinv[:, None]
    dinv_ref[...] = dinv
    dinv2_ref[...] = 1.0 / deg


def _mm_call(x_pad, w, deg_pair):
    bm = 512
    return pl.pallas_call(
        _mm_body,
        grid=(N_PAD // bm,),
        in_specs=[
            pl.BlockSpec((bm, F_IN), lambda i: (i, 0)),
            pl.BlockSpec((F_OUT, F_IN), lambda i: (0, 0)),
            pl.BlockSpec((2, bm), lambda i: (0, i)),
        ],
        out_specs=[
            pl.BlockSpec((bm, F_OUT), lambda i: (i, 0)),
            pl.BlockSpec((bm,), lambda i: (i,)),
            pl.BlockSpec((bm,), lambda i: (i,)),
        ],
        out_shape=[
            jax.ShapeDtypeStruct((N_PAD, F_OUT), jnp.float32),
            jax.ShapeDtypeStruct((N_PAD,), jnp.float32),
            jax.ShapeDtypeStruct((N_PAD,), jnp.float32),
        ],
    )(x_pad, w, deg_pair)


def _mid_body(tp_ref, dinv2_ref, s_ref):
    t = tp_ref[0] + tp_ref[1]
    s_ref[...] = t * dinv2_ref[...][:, None]


def _mid_call(t_pair, dinv2):
    bm = 512
    return pl.pallas_call(
        _mid_body,
        grid=(N_PAD // bm,),
        in_specs=[
            pl.BlockSpec((2, bm, F_OUT), lambda i: (0, i, 0)),
            pl.BlockSpec((bm,), lambda i: (i,)),
        ],
        out_specs=pl.BlockSpec((bm, F_OUT), lambda i: (i, 0)),
        out_shape=jax.ShapeDtypeStruct((N_PAD, F_OUT), jnp.float32),
    )(t_pair, dinv2)


def _fin_body(tp_ref, dinv_ref, b_ref, o_ref):
    t = tp_ref[0] + tp_ref[1]
    o_ref[...] = t * dinv_ref[...][:, None] + b_ref[...][None, :]


def _fin_call(t_pair, dinv, b):
    bm = 512
    return pl.pallas_call(
        _fin_body,
        grid=(N_PAD // bm,),
        in_specs=[
            pl.BlockSpec((2, bm, F_OUT), lambda i: (0, i, 0)),
            pl.BlockSpec((bm,), lambda i: (i,)),
            pl.BlockSpec((F_OUT,), lambda i: (0,)),
        ],
        out_specs=pl.BlockSpec((bm, F_OUT), lambda i: (i, 0)),
        out_shape=jax.ShapeDtypeStruct((N_PAD, F_OUT), jnp.float32),
    )(t_pair, dinv, b)


def kernel(x, adj, W, b):
    src = adj[0].astype(jnp.int32)
    dst = adj[1].astype(jnp.int32)
    padidx = N + (jnp.arange(E_PAD - E, dtype=jnp.int32) % (N_PAD - N))
    src_flat = jnp.concatenate([src, padidx])
    dst_flat = jnp.concatenate([dst, padidx])
    dst2d = dst_flat.reshape(CHUNKS, C)
    src2dg = src_flat.reshape(CHUNKS_G, CG)
    dst2dg = dst_flat.reshape(CHUNKS_G, CG)
    x_pad = jnp.concatenate(
        [x, jnp.zeros((N_PAD - N, F_IN), jnp.float32)], axis=0
    )

    deg_pair = _deg_kernel(dst2d)
    s0, dinv, dinv2 = _mm_call(x_pad, W, deg_pair)
    t0 = _prop_kernel(s0, src2dg, dst2dg)
    s1 = _mid_call(t0, dinv2)
    t1 = _prop_kernel(s1, src2dg, dst2dg)
    out = _fin_call(t1, dinv, b)
    return out[:N]


# speedup vs baseline: 32.7399x; 1.0933x over previous
import functools

import jax
import jax.numpy as jnp
from jax import lax
from jax.experimental import pallas as pl
from jax.experimental.pallas import tpu as pltpu
from jax.experimental.pallas import tpu_sc as plsc

N = 10000
N_PAD = 10240
E = 160000
E_PAD = 163840
F_IN = 256
F_OUT = 128
C = 128
CG = 64
ROWS_T = N_PAD // 16
CHUNKS = E_PAD // C
CPW = CHUNKS // 32
CHUNKS_G = E_PAD // CG
CPWG = CHUNKS_G // 32

_MESH = plsc.VectorSubcoreMesh(core_axis_name="c", subcore_axis_name="s")


@functools.partial(
    pl.kernel,
    out_type=jax.ShapeDtypeStruct((2, N_PAD), jnp.float32),
    mesh=_MESH,
    scratch_types=[
        pltpu.VMEM((CPW, C), jnp.int32),
        pltpu.VMEM((C,), jnp.float32),
        pltpu.VMEM((ROWS_T,), jnp.float32),
        pltpu.VMEM_SHARED((N_PAD,), jnp.float32),
        pltpu.SemaphoreType.DMA,
    ],
)
def _deg_kernel(dst2d, deg_pair, dsts, ones, zer, deg_sh, qsem):
    c = lax.axis_index("c")
    s = lax.axis_index("s")
    wid = c * 16 + s
    row0 = s * ROWS_T
    for i in range(C // 16):
        ones[pl.ds(i * 16, 16)] = jnp.ones((16,), jnp.float32)
    for i in range(ROWS_T // 16):
        zer[pl.ds(i * 16, 16)] = jnp.zeros((16,), jnp.float32)
    pltpu.sync_copy(zer, deg_sh.at[pl.ds(row0, ROWS_T)])
    pltpu.sync_copy(dst2d.at[pl.ds(wid * CPW, CPW)], dsts)
    plsc.subcore_barrier()

    @pl.loop(0, CPW)
    def _(j):
        pltpu.async_copy(ones, deg_sh.at[dsts.at[j]], qsem)

    @pl.loop(0, CPW)
    def _(j):
        pltpu.make_async_copy(ones, deg_sh.at[dsts.at[j]], qsem).wait()

    plsc.subcore_barrier()
    pltpu.sync_copy(deg_sh.at[pl.ds(row0, ROWS_T)], deg_pair.at[c, pl.ds(row0, ROWS_T)])


@functools.partial(
    pl.kernel,
    out_type=jax.ShapeDtypeStruct((2, N_PAD, F_OUT), jnp.float32),
    mesh=_MESH,
    scratch_types=[
        pltpu.VMEM((8, CG), jnp.int32),
        pltpu.VMEM((8, CG), jnp.int32),
        pltpu.VMEM((4, CG, F_OUT), jnp.float32),
        pltpu.VMEM((16, F_OUT), jnp.float32),
        pltpu.VMEM_SHARED((N_PAD, F_OUT), jnp.float32),
        [pltpu.SemaphoreType.DMA] * 4,
        [pltpu.SemaphoreType.DMA] * 8,
        [pltpu.SemaphoreType.DMA] * 8,
    ],
)
def _prop_kernel(
    g, src2d, dst2d, t_pair, sidx, didx, rows, zbuf, acc, gsem, ssem, dsem
):
    c = lax.axis_index("c")
    s = lax.axis_index("s")
    wid = c * 16 + s
    row0 = s * ROWS_T
    ch0 = wid * CPWG

    for k in range(8):
        pltpu.async_copy(src2d.at[pl.ds(ch0 + k, 1)], sidx.at[pl.ds(k, 1)], ssem[k])
        pltpu.async_copy(dst2d.at[pl.ds(ch0 + k, 1)], didx.at[pl.ds(k, 1)], dsem[k])

    @pl.when(c == 0)
    def _():
        pltpu.sync_copy(g.at[pl.ds(row0, ROWS_T)], acc.at[pl.ds(row0, ROWS_T)])

    @pl.when(c == 1)
    def _():
        for i in range(16 * F_OUT // 16):
            zbuf[pl.ds(i // 8, 1), pl.ds((i % 8) * 16, 16)] = jnp.zeros(
                (1, 16), jnp.float32
            )

        @pl.loop(0, ROWS_T // 16)
        def _(j):
            pltpu.sync_copy(zbuf, acc.at[pl.ds(row0 + j * 16, 16)])

    for k in range(4):
        pltpu.make_async_copy(
            src2d.at[pl.ds(ch0 + k, 1)], sidx.at[pl.ds(k, 1)], ssem[k]
        ).wait()
        pltpu.async_copy(g.at[sidx.at[k]], rows.at[k], gsem[k])
    plsc.subcore_barrier()

    @pl.loop(0, CPWG, step=8)
    def _(j):
        for b in range(8):
            rb = b % 4
            pltpu.make_async_copy(g.at[sidx.at[b]], rows.at[rb], gsem[rb]).wait()
            pltpu.make_async_copy(
                dst2d.at[pl.ds(ch0, 1)], didx.at[pl.ds(b, 1)], dsem[b]
            ).wait()
            pltpu.sync_copy(rows.at[rb], acc.at[didx.at[b]], add=True)

            @pl.when(j + b + 8 < CPWG)
            def _():
                pltpu.async_copy(
                    src2d.at[pl.ds(ch0 + j + b + 8, 1)],
                    sidx.at[pl.ds(b, 1)],
                    ssem[b],
                )
                pltpu.async_copy(
                    dst2d.at[pl.ds(ch0 + j + b + 8, 1)],
                    didx.at[pl.ds(b, 1)],
                    dsem[b],
                )

            @pl.when(j + b + 4 < CPWG)
            def _():
                b4 = (b + 4) % 8
                pltpu.make_async_copy(
                    src2d.at[pl.ds(ch0, 1)], sidx.at[pl.ds(b4, 1)], ssem[b4]
                ).wait()
                pltpu.async_copy(g.at[sidx.at[b4]], rows.at[rb], gsem[rb])

    plsc.subcore_barrier()
    pltpu.sync_copy(acc.at[pl.ds(row0, ROWS_T)], t_pair.at[c, pl.ds(row0, ROWS_T)])


def _mm_body(x_ref, w_ref, degp_ref, s0_ref, dinv_ref, dinv2_ref):
    deg = degp_ref[0] + degp_ref[1] + 1.0
    dinv = lax.rsqrt(deg)
    z = jax.lax.dot_general(
        x_ref[...], w_ref[...], (((1,), (1,)), ((), ())),
        preferred_element_type=jnp.float32,
    )
    s0_ref[...] = z * dinv[:, None]
    dinv_ref[...] = dinv
    dinv2_ref[...] = 1.0 / deg


def _mm_call(x_pad, w, deg_pair):
    bm = 1024
    return pl.pallas_call(
        _mm_body,
        grid=(N_PAD // bm,),
        in_specs=[
            pl.BlockSpec((bm, F_IN), lambda i: (i, 0)),
            pl.BlockSpec((F_OUT, F_IN), lambda i: (0, 0)),
            pl.BlockSpec((2, bm), lambda i: (0, i)),
        ],
        out_specs=[
            pl.BlockSpec((bm, F_OUT), lambda i: (i, 0)),
            pl.BlockSpec((bm,), lambda i: (i,)),
            pl.BlockSpec((bm,), lambda i: (i,)),
        ],
        out_shape=[
            jax.ShapeDtypeStruct((N_PAD, F_OUT), jnp.float32),
            jax.ShapeDtypeStruct((N_PAD,), jnp.float32),
            jax.ShapeDtypeStruct((N_PAD,), jnp.float32),
        ],
    )(x_pad, w, deg_pair)


def _mid_body(tp_ref, dinv2_ref, s_ref):
    t = tp_ref[0] + tp_ref[1]
    s_ref[...] = t * dinv2_ref[...][:, None]


def _mid_call(t_pair, dinv2):
    bm = 1024
    return pl.pallas_call(
        _mid_body,
        grid=(N_PAD // bm,),
        in_specs=[
            pl.BlockSpec((2, bm, F_OUT), lambda i: (0, i, 0)),
            pl.BlockSpec((bm,), lambda i: (i,)),
        ],
        out_specs=pl.BlockSpec((bm, F_OUT), lambda i: (i, 0)),
        out_shape=jax.ShapeDtypeStruct((N_PAD, F_OUT), jnp.float32),
    )(t_pair, dinv2)


def _fin_body(tp_ref, dinv_ref, b_ref, o_ref):
    t = tp_ref[0] + tp_ref[1]
    o_ref[...] = t * dinv_ref[...][:, None] + b_ref[...][None, :]


def _fin_call(t_pair, dinv, b):
    bm = 1024
    return pl.pallas_call(
        _fin_body,
        grid=(N_PAD // bm,),
        in_specs=[
            pl.BlockSpec((2, bm, F_OUT), lambda i: (0, i, 0)),
            pl.BlockSpec((bm,), lambda i: (i,)),
            pl.BlockSpec((F_OUT,), lambda i: (0,)),
        ],
        out_specs=pl.BlockSpec((bm, F_OUT), lambda i: (i, 0)),
        out_shape=jax.ShapeDtypeStruct((N_PAD, F_OUT), jnp.float32),
    )(t_pair, dinv, b)


def kernel(x, adj, W, b):
    src = adj[0].astype(jnp.int32)
    dst = adj[1].astype(jnp.int32)
    padidx = N + (jnp.arange(E_PAD - E, dtype=jnp.int32) % (N_PAD - N))
    src_flat = jnp.concatenate([src, padidx])
    dst_flat = jnp.concatenate([dst, padidx])
    dst2d = dst_flat.reshape(CHUNKS, C)
    src2dg = src_flat.reshape(CHUNKS_G, CG)
    dst2dg = dst_flat.reshape(CHUNKS_G, CG)
    x_pad = jnp.concatenate(
        [x, jnp.zeros((N_PAD - N, F_IN), jnp.float32)], axis=0
    )

    deg_pair = _deg_kernel(dst2d)
    s0, dinv, dinv2 = _mm_call(x_pad, W, deg_pair)
    t0 = _prop_kernel(s0, src2dg, dst2dg)
    s1 = _mid_call(t0, dinv2)
    t1 = _prop_kernel(s1, src2dg, dst2dg)
    out = _fin_call(t1, dinv, b)
    return out[:N]


# speedup vs baseline: 32.7981x; 1.0018x over previous
import functools

import jax
import jax.numpy as jnp
from jax import lax
from jax.experimental import pallas as pl
from jax.experimental.pallas import tpu as pltpu
from jax.experimental.pallas import tpu_sc as plsc

N = 10000
N_PAD = 10240
E = 160000
E_PAD = 163840
F_IN = 256
F_OUT = 128
C = 128
CG = 64
ROWS_T = N_PAD // 16
CHUNKS = E_PAD // C
CPW = CHUNKS // 32
CHUNKS_G = E_PAD // CG
CPWG = CHUNKS_G // 32

_MESH = plsc.VectorSubcoreMesh(core_axis_name="c", subcore_axis_name="s")


@functools.partial(
    pl.kernel,
    out_type=jax.ShapeDtypeStruct((2, N_PAD), jnp.float32),
    mesh=_MESH,
    scratch_types=[
        pltpu.VMEM((CPW, C), jnp.int32),
        pltpu.VMEM((C,), jnp.float32),
        pltpu.VMEM((ROWS_T,), jnp.float32),
        pltpu.VMEM_SHARED((N_PAD,), jnp.float32),
        pltpu.SemaphoreType.DMA,
    ],
)
def _deg_kernel(dst2d, deg_pair, dsts, ones, zer, deg_sh, qsem):
    c = lax.axis_index("c")
    s = lax.axis_index("s")
    wid = c * 16 + s
    row0 = s * ROWS_T
    for i in range(C // 16):
        ones[pl.ds(i * 16, 16)] = jnp.ones((16,), jnp.float32)
    for i in range(ROWS_T // 16):
        zer[pl.ds(i * 16, 16)] = jnp.zeros((16,), jnp.float32)
    pltpu.sync_copy(zer, deg_sh.at[pl.ds(row0, ROWS_T)])
    pltpu.sync_copy(dst2d.at[pl.ds(wid * CPW, CPW)], dsts)
    plsc.subcore_barrier()

    @pl.loop(0, CPW)
    def _(j):
        pltpu.sync_copy(ones, deg_sh.at[dsts.at[j]], add=True)

    plsc.subcore_barrier()
    pltpu.sync_copy(deg_sh.at[pl.ds(row0, ROWS_T)], deg_pair.at[c, pl.ds(row0, ROWS_T)])


@functools.partial(
    pl.kernel,
    out_type=jax.ShapeDtypeStruct((2, N_PAD, F_OUT), jnp.float32),
    mesh=_MESH,
    scratch_types=[
        pltpu.VMEM((8, CG), jnp.int32),
        pltpu.VMEM((8, CG), jnp.int32),
        pltpu.VMEM((4, CG, F_OUT), jnp.float32),
        pltpu.VMEM((16, F_OUT), jnp.float32),
        pltpu.VMEM_SHARED((N_PAD, F_OUT), jnp.float32),
        [pltpu.SemaphoreType.DMA] * 4,
        [pltpu.SemaphoreType.DMA] * 8,
        [pltpu.SemaphoreType.DMA] * 8,
    ],
)
def _prop_kernel(
    g, src2d, dst2d, t_pair, sidx, didx, rows, zbuf, acc, gsem, ssem, dsem
):
    c = lax.axis_index("c")
    s = lax.axis_index("s")
    wid = c * 16 + s
    row0 = s * ROWS_T
    ch0 = wid * CPWG

    for k in range(8):
        pltpu.async_copy(src2d.at[pl.ds(ch0 + k, 1)], sidx.at[pl.ds(k, 1)], ssem[k])
        pltpu.async_copy(dst2d.at[pl.ds(ch0 + k, 1)], didx.at[pl.ds(k, 1)], dsem[k])

    @pl.when(c == 0)
    def _():
        pltpu.sync_copy(g.at[pl.ds(row0, ROWS_T)], acc.at[pl.ds(row0, ROWS_T)])

    @pl.when(c == 1)
    def _():
        for i in range(16 * F_OUT // 16):
            zbuf[pl.ds(i // 8, 1), pl.ds((i % 8) * 16, 16)] = jnp.zeros(
                (1, 16), jnp.float32
            )

        @pl.loop(0, ROWS_T // 16)
        def _(j):
            pltpu.sync_copy(zbuf, acc.at[pl.ds(row0 + j * 16, 16)])

    for k in range(4):
        pltpu.make_async_copy(
            src2d.at[pl.ds(ch0 + k, 1)], sidx.at[pl.ds(k, 1)], ssem[k]
        ).wait()
        pltpu.async_copy(g.at[sidx.at[k]], rows.at[k], gsem[k])
    plsc.subcore_barrier()

    @pl.loop(0, CPWG, step=8)
    def _(j):
        for b in range(8):
            rb = b % 4
            pltpu.make_async_copy(g.at[sidx.at[b]], rows.at[rb], gsem[rb]).wait()
            pltpu.make_async_copy(
                dst2d.at[pl.ds(ch0, 1)], didx.at[pl.ds(b, 1)], dsem[b]
            ).wait()
            pltpu.sync_copy(rows.at[rb], acc.at[didx.at[b]], add=True)

            @pl.when(j + b + 8 < CPWG)
            def _():
                pltpu.async_copy(
                    src2d.at[pl.ds(ch0 + j + b + 8, 1)],
                    sidx.at[pl.ds(b, 1)],
                    ssem[b],
                )
                pltpu.async_copy(
                    dst2d.at[pl.ds(ch0 + j + b + 8, 1)],
                    didx.at[pl.ds(b, 1)],
                    dsem[b],
                )

            @pl.when(j + b + 4 < CPWG)
            def _():
                b4 = (b + 4) % 8
                pltpu.make_async_copy(
                    src2d.at[pl.ds(ch0, 1)], sidx.at[pl.ds(b4, 1)], ssem[b4]
                ).wait()
                pltpu.async_copy(g.at[sidx.at[b4]], rows.at[rb], gsem[rb])

    plsc.subcore_barrier()
    pltpu.sync_copy(acc.at[pl.ds(row0, ROWS_T)], t_pair.at[c, pl.ds(row0, ROWS_T)])


def _mm_body(x_ref, w_ref, degp_ref, s0_ref, dinv_ref, dinv2_ref):
    deg = degp_ref[0] + degp_ref[1] + 1.0
    dinv = lax.rsqrt(deg)
    z = jax.lax.dot_general(
        x_ref[...], w_ref[...], (((1,), (1,)), ((), ())),
        preferred_element_type=jnp.float32,
    )
    s0_ref[...] = z * dinv[:, None]
    dinv_ref[...] = dinv
    dinv2_ref[...] = 1.0 / deg


def _mm_call(x_pad, w, deg_pair):
    bm = 1024
    return pl.pallas_call(
        _mm_body,
        grid=(N_PAD // bm,),
        in_specs=[
            pl.BlockSpec((bm, F_IN), lambda i: (i, 0)),
            pl.BlockSpec((F_OUT, F_IN), lambda i: (0, 0)),
            pl.BlockSpec((2, bm), lambda i: (0, i)),
        ],
        out_specs=[
            pl.BlockSpec((bm, F_OUT), lambda i: (i, 0)),
            pl.BlockSpec((bm,), lambda i: (i,)),
            pl.BlockSpec((bm,), lambda i: (i,)),
        ],
        out_shape=[
            jax.ShapeDtypeStruct((N_PAD, F_OUT), jnp.float32),
            jax.ShapeDtypeStruct((N_PAD,), jnp.float32),
            jax.ShapeDtypeStruct((N_PAD,), jnp.float32),
        ],
    )(x_pad, w, deg_pair)


def _mid_body(tp_ref, dinv2_ref, s_ref):
    t = tp_ref[0] + tp_ref[1]
    s_ref[...] = t * dinv2_ref[...][:, None]


def _mid_call(t_pair, dinv2):
    bm = 1024
    return pl.pallas_call(
        _mid_body,
        grid=(N_PAD // bm,),
        in_specs=[
            pl.BlockSpec((2, bm, F_OUT), lambda i: (0, i, 0)),
            pl.BlockSpec((bm,), lambda i: (i,)),
        ],
        out_specs=pl.BlockSpec((bm, F_OUT), lambda i: (i, 0)),
        out_shape=jax.ShapeDtypeStruct((N_PAD, F_OUT), jnp.float32),
    )(t_pair, dinv2)


def _fin_body(tp_ref, dinv_ref, b_ref, o_ref):
    t = tp_ref[0] + tp_ref[1]
    o_ref[...] = t * dinv_ref[...][:, None] + b_ref[...][None, :]


def _fin_call(t_pair, dinv, b):
    bm = 1024
    return pl.pallas_call(
        _fin_body,
        grid=(N_PAD // bm,),
        in_specs=[
            pl.BlockSpec((2, bm, F_OUT), lambda i: (0, i, 0)),
            pl.BlockSpec((bm,), lambda i: (i,)),
            pl.BlockSpec((F_OUT,), lambda i: (0,)),
        ],
        out_specs=pl.BlockSpec((bm, F_OUT), lambda i: (i, 0)),
        out_shape=jax.ShapeDtypeStruct((N_PAD, F_OUT), jnp.float32),
    )(t_pair, dinv, b)


def kernel(x, adj, W, b):
    src = adj[0].astype(jnp.int32)
    dst = adj[1].astype(jnp.int32)
    padidx = N + (jnp.arange(E_PAD - E, dtype=jnp.int32) % (N_PAD - N))
    src_flat = jnp.concatenate([src, padidx])
    dst_flat = jnp.concatenate([dst, padidx])
    dst2d = dst_flat.reshape(CHUNKS, C)
    src2dg = src_flat.reshape(CHUNKS_G, CG)
    dst2dg = dst_flat.reshape(CHUNKS_G, CG)
    x_pad = jnp.concatenate(
        [x, jnp.zeros((N_PAD - N, F_IN), jnp.float32)], axis=0
    )

    deg_pair = _deg_kernel(dst2d)
    s0, dinv, dinv2 = _mm_call(x_pad, W, deg_pair)
    t0 = _prop_kernel(s0, src2dg, dst2dg)
    s1 = _mid_call(t0, dinv2)
    t1 = _prop_kernel(s1, src2dg, dst2dg)
    out = _fin_call(t1, dinv, b)
    return out[:N]


# speedup vs baseline: 34.1330x; 1.0407x over previous
import functools

import jax
import jax.numpy as jnp
from jax import lax
from jax.experimental import pallas as pl
from jax.experimental.pallas import tpu as pltpu
from jax.experimental.pallas import tpu_sc as plsc

N = 10000
N_PAD = 10240
E = 160000
F_IN = 256
F_OUT = 128
CG = 64
CHUNKS = E // CG
ROWS_T = N_PAD // 16
W_BASE = CHUNKS // 32
W_CUT = 32 - (CHUNKS - 32 * W_BASE)
MAX_CH = 80

_MESH = plsc.VectorSubcoreMesh(core_axis_name="c", subcore_axis_name="s")


def _worker_range(c, s):
    wid = c * 16 + s
    ch0 = W_BASE * wid + jnp.maximum(wid - W_CUT, 0)
    n_w = W_BASE + (wid >= W_CUT).astype(jnp.int32)
    return wid, ch0, n_w


@functools.partial(
    pl.kernel,
    out_type=jax.ShapeDtypeStruct((2, N_PAD), jnp.float32),
    mesh=_MESH,
    scratch_types=[
        pltpu.VMEM((8, CG), jnp.int32),
        pltpu.VMEM((CG,), jnp.float32),
        pltpu.VMEM((ROWS_T,), jnp.float32),
        pltpu.VMEM_SHARED((N_PAD,), jnp.float32),
        [pltpu.SemaphoreType.DMA] * 8,
    ],
)
def _deg_kernel(dst1d, deg_pair, didx, ones, zer, deg_sh, dsem):
    c = lax.axis_index("c")
    s = lax.axis_index("s")
    wid, ch0, n_w = _worker_range(c, s)
    row0 = s * ROWS_T
    for k in range(8):
        pltpu.async_copy(
            dst1d.at[pl.ds((ch0 + k) * CG, CG)], didx.at[k], dsem[k]
        )
    for i in range(CG // 16):
        ones[pl.ds(i * 16, 16)] = jnp.ones((16,), jnp.float32)
    for i in range(ROWS_T // 16):
        zer[pl.ds(i * 16, 16)] = jnp.zeros((16,), jnp.float32)
    pltpu.sync_copy(zer, deg_sh.at[pl.ds(row0, ROWS_T)])
    plsc.subcore_barrier()

    @pl.loop(0, MAX_CH, step=8)
    def _(j):
        for b in range(8):

            @pl.when(j + b < n_w)
            def _():
                pltpu.make_async_copy(
                    dst1d.at[pl.ds(ch0 * CG, CG)], didx.at[b], dsem[b]
                ).wait()
                pltpu.sync_copy(ones, deg_sh.at[didx.at[b]], add=True)

                @pl.when(j + b + 8 < n_w)
                def _():
                    pltpu.async_copy(
                        dst1d.at[pl.ds((ch0 + j + b + 8) * CG, CG)],
                        didx.at[b],
                        dsem[b],
                    )

    plsc.subcore_barrier()
    pltpu.sync_copy(deg_sh.at[pl.ds(row0, ROWS_T)], deg_pair.at[c, pl.ds(row0, ROWS_T)])


@functools.partial(
    pl.kernel,
    out_type=jax.ShapeDtypeStruct((2, N_PAD, F_OUT), jnp.float32),
    mesh=_MESH,
    scratch_types=[
        pltpu.VMEM((8, CG), jnp.int32),
        pltpu.VMEM((8, CG), jnp.int32),
        pltpu.VMEM((4, CG, F_OUT), jnp.float32),
        pltpu.VMEM((16, F_OUT), jnp.float32),
        pltpu.VMEM_SHARED((N_PAD, F_OUT), jnp.float32),
        [pltpu.SemaphoreType.DMA] * 4,
        [pltpu.SemaphoreType.DMA] * 8,
        [pltpu.SemaphoreType.DMA] * 8,
    ],
)
def _prop_kernel(
    g, src1d, dst1d, t_pair, sidx, didx, rows, zbuf, acc, gsem, ssem, dsem
):
    c = lax.axis_index("c")
    s = lax.axis_index("s")
    wid, ch0, n_w = _worker_range(c, s)
    row0 = s * ROWS_T

    for k in range(8):
        pltpu.async_copy(src1d.at[pl.ds((ch0 + k) * CG, CG)], sidx.at[k], ssem[k])
        pltpu.async_copy(dst1d.at[pl.ds((ch0 + k) * CG, CG)], didx.at[k], dsem[k])

    @pl.when(c == 0)
    def _():
        pltpu.sync_copy(g.at[pl.ds(row0, ROWS_T)], acc.at[pl.ds(row0, ROWS_T)])

    @pl.when(c == 1)
    def _():
        for i in range(16 * F_OUT // 16):
            zbuf[pl.ds(i // 8, 1), pl.ds((i % 8) * 16, 16)] = jnp.zeros(
                (1, 16), jnp.float32
            )

        @pl.loop(0, ROWS_T // 16)
        def _(j):
            pltpu.sync_copy(zbuf, acc.at[pl.ds(row0 + j * 16, 16)])

    for k in range(4):
        pltpu.make_async_copy(
            src1d.at[pl.ds((ch0 + k) * CG, CG)], sidx.at[k], ssem[k]
        ).wait()
        pltpu.async_copy(g.at[sidx.at[k]], rows.at[k], gsem[k])
    plsc.subcore_barrier()

    @pl.loop(0, MAX_CH, step=8)
    def _(j):
        for b in range(8):
            rb = b % 4

            @pl.when(j + b < n_w)
            def _():
                pltpu.make_async_copy(g.at[sidx.at[b]], rows.at[rb], gsem[rb]).wait()
                pltpu.make_async_copy(
                    dst1d.at[pl.ds(ch0 * CG, CG)], didx.at[b], dsem[b]
                ).wait()
                pltpu.sync_copy(rows.at[rb], acc.at[didx.at[b]], add=True)

                @pl.when(j + b + 8 < n_w)
                def _():
                    pltpu.async_copy(
                        src1d.at[pl.ds((ch0 + j + b + 8) * CG, CG)],
                        sidx.at[b],
                        ssem[b],
                    )
                    pltpu.async_copy(
                        dst1d.at[pl.ds((ch0 + j + b + 8) * CG, CG)],
                        didx.at[b],
                        dsem[b],
                    )

                @pl.when(j + b + 4 < n_w)
                def _():
                    b4 = (b + 4) % 8
                    pltpu.make_async_copy(
                        src1d.at[pl.ds(ch0 * CG, CG)], sidx.at[b4], ssem[b4]
                    ).wait()
                    pltpu.async_copy(g.at[sidx.at[b4]], rows.at[rb], gsem[rb])

    plsc.subcore_barrier()
    pltpu.sync_copy(acc.at[pl.ds(row0, ROWS_T)], t_pair.at[c, pl.ds(row0, ROWS_T)])


def _mm_body(x_ref, w_ref, degp_ref, s0_ref, dinv_ref, dinv2_ref):
    deg = degp_ref[0] + degp_ref[1] + 1.0
    dinv = lax.rsqrt(deg)
    z = jax.lax.dot_general(
        x_ref[...], w_ref[...], (((1,), (1,)), ((), ())),
        preferred_element_type=jnp.float32,
    )
    s0_ref[...] = z * dinv[:, None]
    dinv_ref[...] = dinv
    dinv2_ref[...] = 1.0 / deg


def _mm_call(x, w, deg_pair):
    bm = 1024
    return pl.pallas_call(
        _mm_body,
        grid=(N_PAD // bm,),
        in_specs=[
            pl.BlockSpec((bm, F_IN), lambda i: (i, 0)),
            pl.BlockSpec((F_OUT, F_IN), lambda i: (0, 0)),
            pl.BlockSpec((2, bm), lambda i: (0, i)),
        ],
        out_specs=[
            pl.BlockSpec((bm, F_OUT), lambda i: (i, 0)),
            pl.BlockSpec((bm,), lambda i: (i,)),
            pl.BlockSpec((bm,), lambda i: (i,)),
        ],
        out_shape=[
            jax.ShapeDtypeStruct((N_PAD, F_OUT), jnp.float32),
            jax.ShapeDtypeStruct((N_PAD,), jnp.float32),
            jax.ShapeDtypeStruct((N_PAD,), jnp.float32),
        ],
    )(x, w, deg_pair)


def _mid_body(tp_ref, dinv2_ref, s_ref):
    t = tp_ref[0] + tp_ref[1]
    s_ref[...] = t * dinv2_ref[...][:, None]


def _mid_call(t_pair, dinv2):
    bm = 1024
    return pl.pallas_call(
        _mid_body,
        grid=(N_PAD // bm,),
        in_specs=[
            pl.BlockSpec((2, bm, F_OUT), lambda i: (0, i, 0)),
            pl.BlockSpec((bm,), lambda i: (i,)),
        ],
        out_specs=pl.BlockSpec((bm, F_OUT), lambda i: (i, 0)),
        out_shape=jax.ShapeDtypeStruct((N_PAD, F_OUT), jnp.float32),
    )(t_pair, dinv2)


def _fin_body(tp_ref, dinv_ref, b_ref, o_ref):
    t = tp_ref[0] + tp_ref[1]
    o_ref[...] = t * dinv_ref[...][:, None] + b_ref[...][None, :]


def _fin_call(t_pair, dinv, b):
    bm = 1024
    return pl.pallas_call(
        _fin_body,
        grid=(N_PAD // bm,),
        in_specs=[
            pl.BlockSpec((2, bm, F_OUT), lambda i: (0, i, 0)),
            pl.BlockSpec((bm,), lambda i: (i,)),
            pl.BlockSpec((F_OUT,), lambda i: (0,)),
        ],
        out_specs=pl.BlockSpec((bm, F_OUT), lambda i: (i, 0)),
        out_shape=jax.ShapeDtypeStruct((N, F_OUT), jnp.float32),
    )(t_pair, dinv, b)


def kernel(x, adj, W, b):
    adji = adj.astype(jnp.int32)
    src1d = adji[0]
    dst1d = adji[1]

    deg_pair = _deg_kernel(dst1d)
    s0, dinv, dinv2 = _mm_call(x, W, deg_pair)
    t0 = _prop_kernel(s0, src1d, dst1d)
    s1 = _mid_call(t0, dinv2)
    t1 = _prop_kernel(s1, src1d, dst1d)
    return _fin_call(t1, dinv, b)


# speedup vs baseline: 36.0819x; 1.0571x over previous
import functools

import jax
import jax.numpy as jnp
from jax import lax
from jax.experimental import pallas as pl
from jax.experimental.pallas import tpu as pltpu
from jax.experimental.pallas import tpu_sc as plsc

N = 10000
N_PAD = 10240
E = 160000
F_IN = 256
F_OUT = 128
CG = 64
CHUNKS = E // CG
ROWS_T = N_PAD // 16
W_BASE = CHUNKS // 32
W_CUT = 32 - (CHUNKS - 32 * W_BASE)
MAX_CH = 80

_MESH = plsc.VectorSubcoreMesh(core_axis_name="c", subcore_axis_name="s")


def _worker_range(c, s):
    wid = c * 16 + s
    ch0 = W_BASE * wid + jnp.maximum(wid - W_CUT, 0)
    n_w = W_BASE + (wid >= W_CUT).astype(jnp.int32)
    return wid, ch0, n_w


@functools.partial(
    pl.kernel,
    out_type=jax.ShapeDtypeStruct((2, N_PAD), jnp.float32),
    mesh=_MESH,
    scratch_types=[
        pltpu.VMEM((8, CG), jnp.int32),
        pltpu.VMEM((CG,), jnp.float32),
        pltpu.VMEM((ROWS_T,), jnp.float32),
        pltpu.VMEM_SHARED((N_PAD,), jnp.float32),
        [pltpu.SemaphoreType.DMA] * 8,
    ],
)
def _deg_kernel(dst1d, deg_pair, didx, ones, zer, deg_sh, dsem):
    c = lax.axis_index("c")
    s = lax.axis_index("s")
    wid, ch0, n_w = _worker_range(c, s)
    row0 = s * ROWS_T
    for k in range(8):
        pltpu.async_copy(
            dst1d.at[pl.ds((ch0 + k) * CG, CG)], didx.at[k], dsem[k]
        )
    for i in range(CG // 16):
        ones[pl.ds(i * 16, 16)] = jnp.ones((16,), jnp.float32)
    for i in range(ROWS_T // 16):
        zer[pl.ds(i * 16, 16)] = jnp.zeros((16,), jnp.float32)
    pltpu.sync_copy(zer, deg_sh.at[pl.ds(row0, ROWS_T)])
    plsc.subcore_barrier()

    @pl.loop(0, MAX_CH, step=8)
    def _(j):
        for b in range(8):

            @pl.when(j + b < n_w)
            def _():
                pltpu.make_async_copy(
                    dst1d.at[pl.ds(ch0 * CG, CG)], didx.at[b], dsem[b]
                ).wait()
                pltpu.sync_copy(ones, deg_sh.at[didx.at[b]], add=True)

                @pl.when(j + b + 8 < n_w)
                def _():
                    pltpu.async_copy(
                        dst1d.at[pl.ds((ch0 + j + b + 8) * CG, CG)],
                        didx.at[b],
                        dsem[b],
                    )

    plsc.subcore_barrier()
    pltpu.sync_copy(deg_sh.at[pl.ds(row0, ROWS_T)], deg_pair.at[c, pl.ds(row0, ROWS_T)])


@functools.partial(
    pl.kernel,
    out_type=jax.ShapeDtypeStruct((2, N_PAD, F_OUT), jnp.float32),
    mesh=_MESH,
    scratch_types=[
        pltpu.VMEM((8, CG), jnp.int32),
        pltpu.VMEM((8, CG), jnp.int32),
        pltpu.VMEM((4, CG, F_OUT), jnp.float32),
        pltpu.VMEM((16, F_OUT), jnp.float32),
        pltpu.VMEM_SHARED((N_PAD, F_OUT), jnp.float32),
        [pltpu.SemaphoreType.DMA] * 4,
        [pltpu.SemaphoreType.DMA] * 8,
        [pltpu.SemaphoreType.DMA] * 8,
    ],
)
def _prop_kernel(
    g, src1d, dst1d, t_pair, sidx, didx, rows, zbuf, acc, gsem, ssem, dsem
):
    c = lax.axis_index("c")
    s = lax.axis_index("s")
    wid, ch0, n_w = _worker_range(c, s)
    row0 = s * ROWS_T

    for k in range(8):
        pltpu.async_copy(src1d.at[pl.ds((ch0 + k) * CG, CG)], sidx.at[k], ssem[k])
        pltpu.async_copy(dst1d.at[pl.ds((ch0 + k) * CG, CG)], didx.at[k], dsem[k])

    @pl.when(c == 0)
    def _():
        pltpu.sync_copy(g.at[pl.ds(row0, ROWS_T)], acc.at[pl.ds(row0, ROWS_T)])

    @pl.when(c == 1)
    def _():
        for i in range(16 * F_OUT // 16):
            zbuf[pl.ds(i // 8, 1), pl.ds((i % 8) * 16, 16)] = jnp.zeros(
                (1, 16), jnp.float32
            )

        @pl.loop(0, ROWS_T // 16)
        def _(j):
            pltpu.sync_copy(zbuf, acc.at[pl.ds(row0 + j * 16, 16)])

    for k in range(4):
        pltpu.make_async_copy(
            src1d.at[pl.ds((ch0 + k) * CG, CG)], sidx.at[k], ssem[k]
        ).wait()
        pltpu.async_copy(g.at[sidx.at[k]], rows.at[k], gsem[k])
    plsc.subcore_barrier()

    @pl.loop(0, MAX_CH, step=8)
    def _(j):
        for b in range(8):
            rb = b % 4

            @pl.when(j + b < n_w)
            def _():
                pltpu.make_async_copy(g.at[sidx.at[b]], rows.at[rb], gsem[rb]).wait()
                pltpu.make_async_copy(
                    dst1d.at[pl.ds(ch0 * CG, CG)], didx.at[b], dsem[b]
                ).wait()
                pltpu.sync_copy(rows.at[rb], acc.at[didx.at[b]], add=True)

                @pl.when(j + b + 8 < n_w)
                def _():
                    pltpu.async_copy(
                        src1d.at[pl.ds((ch0 + j + b + 8) * CG, CG)],
                        sidx.at[b],
                        ssem[b],
                    )
                    pltpu.async_copy(
                        dst1d.at[pl.ds((ch0 + j + b + 8) * CG, CG)],
                        didx.at[b],
                        dsem[b],
                    )

                @pl.when(j + b + 4 < n_w)
                def _():
                    b4 = (b + 4) % 8
                    pltpu.make_async_copy(
                        src1d.at[pl.ds(ch0 * CG, CG)], sidx.at[b4], ssem[b4]
                    ).wait()
                    pltpu.async_copy(g.at[sidx.at[b4]], rows.at[rb], gsem[rb])

    plsc.subcore_barrier()
    pltpu.sync_copy(acc.at[pl.ds(row0, ROWS_T)], t_pair.at[c, pl.ds(row0, ROWS_T)])


def _split_body(adj_ref, s_ref, d_ref):
    sd = adj_ref[...]
    s_ref[...] = sd[0]
    d_ref[...] = sd[1]


def _split_call(adj):
    bk = 16384
    return pl.pallas_call(
        _split_body,
        grid=((E + bk - 1) // bk,),
        in_specs=[pl.BlockSpec((2, bk), lambda i: (0, i))],
        out_specs=[
            pl.BlockSpec((bk,), lambda i: (i,)),
            pl.BlockSpec((bk,), lambda i: (i,)),
        ],
        out_shape=[
            jax.ShapeDtypeStruct((E,), jnp.int32),
            jax.ShapeDtypeStruct((E,), jnp.int32),
        ],
    )(adj)


def _zmm_body(x_ref, w_ref, z_ref):
    z_ref[...] = jax.lax.dot_general(
        x_ref[...], w_ref[...], (((1,), (1,)), ((), ())),
        preferred_element_type=jnp.float32,
    )


def _zmm_call(x, w):
    bm = 2048
    return pl.pallas_call(
        _zmm_body,
        grid=(N_PAD // bm,),
        in_specs=[
            pl.BlockSpec((bm, F_IN), lambda i: (i, 0)),
            pl.BlockSpec((F_OUT, F_IN), lambda i: (0, 0)),
        ],
        out_specs=pl.BlockSpec((bm, F_OUT), lambda i: (i, 0)),
        out_shape=jax.ShapeDtypeStruct((N_PAD, F_OUT), jnp.float32),
    )(x, w)


def _scale_body(z_ref, degp_ref, s0_ref, dinv_ref, dinv2_ref):
    deg = degp_ref[0] + degp_ref[1] + 1.0
    dinv = lax.rsqrt(deg)
    s0_ref[...] = z_ref[...] * dinv[:, None]
    dinv_ref[...] = dinv
    dinv2_ref[...] = 1.0 / deg


def _scale_call(z, deg_pair):
    bm = 2048
    return pl.pallas_call(
        _scale_body,
        grid=(N_PAD // bm,),
        in_specs=[
            pl.BlockSpec((bm, F_OUT), lambda i: (i, 0)),
            pl.BlockSpec((2, bm), lambda i: (0, i)),
        ],
        out_specs=[
            pl.BlockSpec((bm, F_OUT), lambda i: (i, 0)),
            pl.BlockSpec((bm,), lambda i: (i,)),
            pl.BlockSpec((bm,), lambda i: (i,)),
        ],
        out_shape=[
            jax.ShapeDtypeStruct((N_PAD, F_OUT), jnp.float32),
            jax.ShapeDtypeStruct((N_PAD,), jnp.float32),
            jax.ShapeDtypeStruct((N_PAD,), jnp.float32),
        ],
    )(z, deg_pair)


def _mid_body(tp_ref, dinv2_ref, s_ref):
    t = tp_ref[0] + tp_ref[1]
    s_ref[...] = t * dinv2_ref[...][:, None]


def _mid_call(t_pair, dinv2):
    bm = 2048
    return pl.pallas_call(
        _mid_body,
        grid=(N_PAD // bm,),
        in_specs=[
            pl.BlockSpec((2, bm, F_OUT), lambda i: (0, i, 0)),
            pl.BlockSpec((bm,), lambda i: (i,)),
        ],
        out_specs=pl.BlockSpec((bm, F_OUT), lambda i: (i, 0)),
        out_shape=jax.ShapeDtypeStruct((N_PAD, F_OUT), jnp.float32),
    )(t_pair, dinv2)


def _fin_body(tp_ref, dinv_ref, b_ref, o_ref):
    t = tp_ref[0] + tp_ref[1]
    o_ref[...] = t * dinv_ref[...][:, None] + b_ref[...][None, :]


def _fin_call(t_pair, dinv, b):
    bm = 2048
    return pl.pallas_call(
        _fin_body,
        grid=(N_PAD // bm,),
        in_specs=[
            pl.BlockSpec((2, bm, F_OUT), lambda i: (0, i, 0)),
            pl.BlockSpec((bm,), lambda i: (i,)),
            pl.BlockSpec((F_OUT,), lambda i: (0,)),
        ],
        out_specs=pl.BlockSpec((bm, F_OUT), lambda i: (i, 0)),
        out_shape=jax.ShapeDtypeStruct((N, F_OUT), jnp.float32),
    )(t_pair, dinv, b)


def kernel(x, adj, W, b):
    src1d, dst1d = _split_call(adj.astype(jnp.int32))

    deg_pair = _deg_kernel(dst1d)
    z = _zmm_call(x, W)
    s0, dinv, dinv2 = _scale_call(z, deg_pair)
    t0 = _prop_kernel(s0, src1d, dst1d)
    s1 = _mid_call(t0, dinv2)
    t1 = _prop_kernel(s1, src1d, dst1d)
    return _fin_call(t1, dinv, b)


# speedup vs baseline: 36.7666x; 1.0190x over previous
import functools

import jax
import jax.numpy as jnp
from jax import lax
from jax.experimental import pallas as pl
from jax.experimental.pallas import tpu as pltpu
from jax.experimental.pallas import tpu_sc as plsc

N = 10000
N_PAD = 10240
E = 160000
F_IN = 256
F_OUT = 128
CG = 64
CHUNKS = E // CG
ROWS_T = N_PAD // 16
W_BASE = CHUNKS // 32
W_CUT = 32 - (CHUNKS - 32 * W_BASE)
MAX_CH = 80

_MESH = plsc.VectorSubcoreMesh(core_axis_name="c", subcore_axis_name="s")


def _worker_range(c, s):
    wid = c * 16 + s
    ch0 = W_BASE * wid + jnp.maximum(wid - W_CUT, 0)
    n_w = W_BASE + (wid >= W_CUT).astype(jnp.int32)
    return wid, ch0, n_w


@functools.partial(
    pl.kernel,
    out_type=jax.ShapeDtypeStruct((2, N_PAD), jnp.float32),
    mesh=_MESH,
    scratch_types=[
        pltpu.VMEM((8, CG), jnp.int32),
        pltpu.VMEM((CG,), jnp.float32),
        pltpu.VMEM((ROWS_T,), jnp.float32),
        pltpu.VMEM_SHARED((N_PAD,), jnp.float32),
        [pltpu.SemaphoreType.DMA] * 8,
    ],
)
def _deg_kernel(adj2, deg_pair, didx, ones, zer, deg_sh, dsem):
    dst1d = adj2.at[1]
    c = lax.axis_index("c")
    s = lax.axis_index("s")
    wid, ch0, n_w = _worker_range(c, s)
    row0 = s * ROWS_T
    for k in range(8):
        pltpu.async_copy(
            dst1d.at[pl.ds((ch0 + k) * CG, CG)], didx.at[k], dsem[k]
        )
    for i in range(CG // 16):
        ones[pl.ds(i * 16, 16)] = jnp.ones((16,), jnp.float32)
    for i in range(ROWS_T // 16):
        zer[pl.ds(i * 16, 16)] = jnp.zeros((16,), jnp.float32)
    pltpu.sync_copy(zer, deg_sh.at[pl.ds(row0, ROWS_T)])
    plsc.subcore_barrier()

    @pl.loop(0, MAX_CH, step=8)
    def _(j):
        for b in range(8):

            @pl.when(j + b < n_w)
            def _():
                pltpu.make_async_copy(
                    dst1d.at[pl.ds(ch0 * CG, CG)], didx.at[b], dsem[b]
                ).wait()
                pltpu.sync_copy(ones, deg_sh.at[didx.at[b]], add=True)

                @pl.when(j + b + 8 < n_w)
                def _():
                    pltpu.async_copy(
                        dst1d.at[pl.ds((ch0 + j + b + 8) * CG, CG)],
                        didx.at[b],
                        dsem[b],
                    )

    plsc.subcore_barrier()
    pltpu.sync_copy(deg_sh.at[pl.ds(row0, ROWS_T)], deg_pair.at[c, pl.ds(row0, ROWS_T)])


@functools.partial(
    pl.kernel,
    out_type=jax.ShapeDtypeStruct((2, N_PAD, F_OUT), jnp.float32),
    mesh=_MESH,
    scratch_types=[
        pltpu.VMEM((8, CG), jnp.int32),
        pltpu.VMEM((8, CG), jnp.int32),
        pltpu.VMEM((4, CG, F_OUT), jnp.float32),
        pltpu.VMEM((16, F_OUT), jnp.float32),
        pltpu.VMEM_SHARED((N_PAD, F_OUT), jnp.float32),
        [pltpu.SemaphoreType.DMA] * 4,
        [pltpu.SemaphoreType.DMA] * 8,
        [pltpu.SemaphoreType.DMA] * 8,
    ],
)
def _prop_kernel(
    g, adj2, t_pair, sidx, didx, rows, zbuf, acc, gsem, ssem, dsem
):
    src1d = adj2.at[0]
    dst1d = adj2.at[1]
    c = lax.axis_index("c")
    s = lax.axis_index("s")
    wid, ch0, n_w = _worker_range(c, s)
    row0 = s * ROWS_T

    for k in range(8):
        pltpu.async_copy(src1d.at[pl.ds((ch0 + k) * CG, CG)], sidx.at[k], ssem[k])
        pltpu.async_copy(dst1d.at[pl.ds((ch0 + k) * CG, CG)], didx.at[k], dsem[k])

    @pl.when(c == 0)
    def _():
        pltpu.sync_copy(g.at[pl.ds(row0, ROWS_T)], acc.at[pl.ds(row0, ROWS_T)])

    @pl.when(c == 1)
    def _():
        for i in range(16 * F_OUT // 16):
            zbuf[pl.ds(i // 8, 1), pl.ds((i % 8) * 16, 16)] = jnp.zeros(
                (1, 16), jnp.float32
            )

        @pl.loop(0, ROWS_T // 16)
        def _(j):
            pltpu.sync_copy(zbuf, acc.at[pl.ds(row0 + j * 16, 16)])

    for k in range(4):
        pltpu.make_async_copy(
            src1d.at[pl.ds((ch0 + k) * CG, CG)], sidx.at[k], ssem[k]
        ).wait()
        pltpu.async_copy(g.at[sidx.at[k]], rows.at[k], gsem[k])
    plsc.subcore_barrier()

    @pl.loop(0, MAX_CH, step=8)
    def _(j):
        for b in range(8):
            rb = b % 4

            @pl.when(j + b < n_w)
            def _():
                pltpu.make_async_copy(g.at[sidx.at[b]], rows.at[rb], gsem[rb]).wait()
                pltpu.make_async_copy(
                    dst1d.at[pl.ds(ch0 * CG, CG)], didx.at[b], dsem[b]
                ).wait()
                pltpu.sync_copy(rows.at[rb], acc.at[didx.at[b]], add=True)

                @pl.when(j + b + 8 < n_w)
                def _():
                    pltpu.async_copy(
                        src1d.at[pl.ds((ch0 + j + b + 8) * CG, CG)],
                        sidx.at[b],
                        ssem[b],
                    )
                    pltpu.async_copy(
                        dst1d.at[pl.ds((ch0 + j + b + 8) * CG, CG)],
                        didx.at[b],
                        dsem[b],
                    )

                @pl.when(j + b + 4 < n_w)
                def _():
                    b4 = (b + 4) % 8
                    pltpu.make_async_copy(
                        src1d.at[pl.ds(ch0 * CG, CG)], sidx.at[b4], ssem[b4]
                    ).wait()
                    pltpu.async_copy(g.at[sidx.at[b4]], rows.at[rb], gsem[rb])

    plsc.subcore_barrier()
    pltpu.sync_copy(acc.at[pl.ds(row0, ROWS_T)], t_pair.at[c, pl.ds(row0, ROWS_T)])


def _split_body(adj_ref, s_ref, d_ref):
    sd = adj_ref[...]
    s_ref[...] = sd[0]
    d_ref[...] = sd[1]


def _split_call(adj):
    bk = 16384
    return pl.pallas_call(
        _split_body,
        grid=((E + bk - 1) // bk,),
        in_specs=[pl.BlockSpec((2, bk), lambda i: (0, i))],
        out_specs=[
            pl.BlockSpec((bk,), lambda i: (i,)),
            pl.BlockSpec((bk,), lambda i: (i,)),
        ],
        out_shape=[
            jax.ShapeDtypeStruct((E,), jnp.int32),
            jax.ShapeDtypeStruct((E,), jnp.int32),
        ],
    )(adj)


def _zmm_body(x_ref, w_ref, z_ref):
    z_ref[...] = jax.lax.dot_general(
        x_ref[...], w_ref[...], (((1,), (1,)), ((), ())),
        preferred_element_type=jnp.float32,
    )


def _zmm_call(x, w):
    bm = 2048
    return pl.pallas_call(
        _zmm_body,
        grid=(N_PAD // bm,),
        in_specs=[
            pl.BlockSpec((bm, F_IN), lambda i: (i, 0)),
            pl.BlockSpec((F_OUT, F_IN), lambda i: (0, 0)),
        ],
        out_specs=pl.BlockSpec((bm, F_OUT), lambda i: (i, 0)),
        out_shape=jax.ShapeDtypeStruct((N_PAD, F_OUT), jnp.float32),
    )(x, w)


def _scale_body(z_ref, degp_ref, s0_ref, dinv_ref, dinv2_ref):
    deg = degp_ref[0] + degp_ref[1] + 1.0
    dinv = lax.rsqrt(deg)
    s0_ref[...] = z_ref[...] * dinv[:, None]
    dinv_ref[...] = dinv
    dinv2_ref[...] = 1.0 / deg


def _scale_call(z, deg_pair):
    bm = 2048
    return pl.pallas_call(
        _scale_body,
        grid=(N_PAD // bm,),
        in_specs=[
            pl.BlockSpec((bm, F_OUT), lambda i: (i, 0)),
            pl.BlockSpec((2, bm), lambda i: (0, i)),
        ],
        out_specs=[
            pl.BlockSpec((bm, F_OUT), lambda i: (i, 0)),
            pl.BlockSpec((bm,), lambda i: (i,)),
            pl.BlockSpec((bm,), lambda i: (i,)),
        ],
        out_shape=[
            jax.ShapeDtypeStruct((N_PAD, F_OUT), jnp.float32),
            jax.ShapeDtypeStruct((N_PAD,), jnp.float32),
            jax.ShapeDtypeStruct((N_PAD,), jnp.float32),
        ],
    )(z, deg_pair)


def _mid_body(tp_ref, dinv2_ref, s_ref):
    t = tp_ref[0] + tp_ref[1]
    s_ref[...] = t * dinv2_ref[...][:, None]


def _mid_call(t_pair, dinv2):
    bm = 2048
    return pl.pallas_call(
        _mid_body,
        grid=(N_PAD // bm,),
        in_specs=[
            pl.BlockSpec((2, bm, F_OUT), lambda i: (0, i, 0)),
            pl.BlockSpec((bm,), lambda i: (i,)),
        ],
        out_specs=pl.BlockSpec((bm, F_OUT), lambda i: (i, 0)),
        out_shape=jax.ShapeDtypeStruct((N_PAD, F_OUT), jnp.float32),
    )(t_pair, dinv2)


def _fin_body(tp_ref, dinv_ref, b_ref, o_ref):
    t = tp_ref[0] + tp_ref[1]
    o_ref[...] = t * dinv_ref[...][:, None] + b_ref[...][None, :]


def _fin_call(t_pair, dinv, b):
    bm = 2048
    return pl.pallas_call(
        _fin_body,
        grid=(N_PAD // bm,),
        in_specs=[
            pl.BlockSpec((2, bm, F_OUT), lambda i: (0, i, 0)),
            pl.BlockSpec((bm,), lambda i: (i,)),
            pl.BlockSpec((F_OUT,), lambda i: (0,)),
        ],
        out_specs=pl.BlockSpec((bm, F_OUT), lambda i: (i, 0)),
        out_shape=jax.ShapeDtypeStruct((N, F_OUT), jnp.float32),
    )(t_pair, dinv, b)


def kernel(x, adj, W, b):
    adj2 = adj.astype(jnp.int32)

    deg_pair = _deg_kernel(adj2)
    z = _zmm_call(x, W)
    s0, dinv, dinv2 = _scale_call(z, deg_pair)
    t0 = _prop_kernel(s0, adj2)
    s1 = _mid_call(t0, dinv2)
    t1 = _prop_kernel(s1, adj2)
    return _fin_call(t1, dinv, b)


# speedup vs baseline: 36.7832x; 1.0004x over previous
import functools

import jax
import jax.numpy as jnp
from jax import lax
from jax.experimental import pallas as pl
from jax.experimental.pallas import tpu as pltpu
from jax.experimental.pallas import tpu_sc as plsc

N = 10000
N_PAD = 10240
E = 160000
F_IN = 256
F_OUT = 128
CG = 64
CHUNKS = E // CG
ROWS_T = N_PAD // 16
W_BASE = CHUNKS // 32
W_CUT = 32 - (CHUNKS - 32 * W_BASE)
MAX_CH = 80

_MESH = plsc.VectorSubcoreMesh(core_axis_name="c", subcore_axis_name="s")


def _worker_range(c, s):
    wid = c * 16 + s
    ch0 = W_BASE * wid + jnp.maximum(wid - W_CUT, 0)
    n_w = W_BASE + (wid >= W_CUT).astype(jnp.int32)
    return wid, ch0, n_w


@functools.partial(
    pl.kernel,
    out_type=jax.ShapeDtypeStruct((2, N_PAD), jnp.float32),
    mesh=_MESH,
    scratch_types=[
        pltpu.VMEM((8, CG), jnp.int32),
        pltpu.VMEM((CG,), jnp.float32),
        pltpu.VMEM((ROWS_T,), jnp.float32),
        pltpu.VMEM_SHARED((N_PAD,), jnp.float32),
        [pltpu.SemaphoreType.DMA] * 8,
    ],
)
def _deg_kernel(adj2, deg_pair, didx, ones, zer, deg_sh, dsem):
    dst1d = adj2.at[1]
    c = lax.axis_index("c")
    s = lax.axis_index("s")
    wid, ch0, n_w = _worker_range(c, s)
    row0 = s * ROWS_T
    for k in range(8):
        pltpu.async_copy(
            dst1d.at[pl.ds((ch0 + k) * CG, CG)], didx.at[k], dsem[k]
        )
    for i in range(CG // 16):
        ones[pl.ds(i * 16, 16)] = jnp.ones((16,), jnp.float32)
    for i in range(ROWS_T // 16):
        zer[pl.ds(i * 16, 16)] = jnp.zeros((16,), jnp.float32)
    pltpu.sync_copy(zer, deg_sh.at[pl.ds(row0, ROWS_T)])
    plsc.subcore_barrier()

    @pl.loop(0, MAX_CH, step=8)
    def _(j):
        for b in range(8):

            @pl.when(j + b < n_w)
            def _():
                pltpu.make_async_copy(
                    dst1d.at[pl.ds(ch0 * CG, CG)], didx.at[b], dsem[b]
                ).wait()
                pltpu.sync_copy(ones, deg_sh.at[didx.at[b]], add=True)

                @pl.when(j + b + 8 < n_w)
                def _():
                    pltpu.async_copy(
                        dst1d.at[pl.ds((ch0 + j + b + 8) * CG, CG)],
                        didx.at[b],
                        dsem[b],
                    )

    plsc.subcore_barrier()
    pltpu.sync_copy(deg_sh.at[pl.ds(row0, ROWS_T)], deg_pair.at[c, pl.ds(row0, ROWS_T)])


@functools.partial(
    pl.kernel,
    out_type=jax.ShapeDtypeStruct((2, N_PAD, F_OUT), jnp.float32),
    mesh=_MESH,
    scratch_types=[
        pltpu.VMEM((8, CG), jnp.int32),
        pltpu.VMEM((8, CG), jnp.int32),
        pltpu.VMEM((4, CG, F_OUT), jnp.float32),
        pltpu.VMEM((16, F_OUT), jnp.float32),
        pltpu.VMEM_SHARED((N_PAD, F_OUT), jnp.float32),
        [pltpu.SemaphoreType.DMA] * 4,
        [pltpu.SemaphoreType.DMA] * 8,
        [pltpu.SemaphoreType.DMA] * 8,
    ],
)
def _prop_kernel(
    g, adj2, t_pair, sidx, didx, rows, zbuf, acc, gsem, ssem, dsem
):
    src1d = adj2.at[0]
    dst1d = adj2.at[1]
    c = lax.axis_index("c")
    s = lax.axis_index("s")
    wid, ch0, n_w = _worker_range(c, s)
    row0 = s * ROWS_T

    for k in range(8):
        pltpu.async_copy(src1d.at[pl.ds((ch0 + k) * CG, CG)], sidx.at[k], ssem[k])
        pltpu.async_copy(dst1d.at[pl.ds((ch0 + k) * CG, CG)], didx.at[k], dsem[k])

    @pl.when(c == 0)
    def _():
        pltpu.sync_copy(g.at[pl.ds(row0, ROWS_T)], acc.at[pl.ds(row0, ROWS_T)])

    @pl.when(c == 1)
    def _():
        for i in range(16 * F_OUT // 16):
            zbuf[pl.ds(i // 8, 1), pl.ds((i % 8) * 16, 16)] = jnp.zeros(
                (1, 16), jnp.float32
            )

        @pl.loop(0, ROWS_T // 16)
        def _(j):
            pltpu.sync_copy(zbuf, acc.at[pl.ds(row0 + j * 16, 16)])

    for k in range(4):
        pltpu.make_async_copy(
            src1d.at[pl.ds((ch0 + k) * CG, CG)], sidx.at[k], ssem[k]
        ).wait()
        pltpu.async_copy(g.at[sidx.at[k]], rows.at[k], gsem[k])
    plsc.subcore_barrier()

    @pl.loop(0, MAX_CH, step=8)
    def _(j):
        for b in range(8):
            rb = b % 4

            @pl.when(j + b < n_w)
            def _():
                pltpu.make_async_copy(g.at[sidx.at[b]], rows.at[rb], gsem[rb]).wait()
                pltpu.make_async_copy(
                    dst1d.at[pl.ds(ch0 * CG, CG)], didx.at[b], dsem[b]
                ).wait()
                pltpu.sync_copy(rows.at[rb], acc.at[didx.at[b]], add=True)

                @pl.when(j + b + 8 < n_w)
                def _():
                    pltpu.async_copy(
                        src1d.at[pl.ds((ch0 + j + b + 8) * CG, CG)],
                        sidx.at[b],
                        ssem[b],
                    )
                    pltpu.async_copy(
                        dst1d.at[pl.ds((ch0 + j + b + 8) * CG, CG)],
                        didx.at[b],
                        dsem[b],
                    )

                @pl.when(j + b + 4 < n_w)
                def _():
                    b4 = (b + 4) % 8
                    pltpu.make_async_copy(
                        src1d.at[pl.ds(ch0 * CG, CG)], sidx.at[b4], ssem[b4]
                    ).wait()
                    pltpu.async_copy(g.at[sidx.at[b4]], rows.at[rb], gsem[rb])

    plsc.subcore_barrier()
    pltpu.sync_copy(acc.at[pl.ds(row0, ROWS_T)], t_pair.at[c, pl.ds(row0, ROWS_T)])


def _zmm_body(x_ref, w_ref, z_ref):
    z_ref[...] = jax.lax.dot_general(
        x_ref[...], w_ref[...], (((1,), (1,)), ((), ())),
        preferred_element_type=jnp.float32,
    )


def _zmm_call(x, w):
    bm = 2048
    return pl.pallas_call(
        _zmm_body,
        grid=(N_PAD // bm,),
        in_specs=[
            pl.BlockSpec((bm, F_IN), lambda i: (i, 0)),
            pl.BlockSpec((F_OUT, F_IN), lambda i: (0, 0)),
        ],
        out_specs=pl.BlockSpec((bm, F_OUT), lambda i: (i, 0)),
        out_shape=jax.ShapeDtypeStruct((N_PAD, F_OUT), jnp.float32),
    )(x, w)


def _scale_body(z_ref, degp_ref, s0_ref, dinv_ref, dinv2_ref):
    deg = degp_ref[0] + degp_ref[1] + 1.0
    dinv = lax.rsqrt(deg)
    s0_ref[...] = z_ref[...] * dinv[:, None]
    dinv_ref[...] = dinv
    dinv2_ref[...] = 1.0 / deg


def _scale_call(z, deg_pair):
    bm = 2048
    return pl.pallas_call(
        _scale_body,
        grid=(N_PAD // bm,),
        in_specs=[
            pl.BlockSpec((bm, F_OUT), lambda i: (i, 0)),
            pl.BlockSpec((2, bm), lambda i: (0, i)),
        ],
        out_specs=[
            pl.BlockSpec((bm, F_OUT), lambda i: (i, 0)),
            pl.BlockSpec((bm,), lambda i: (i,)),
            pl.BlockSpec((bm,), lambda i: (i,)),
        ],
        out_shape=[
            jax.ShapeDtypeStruct((N_PAD, F_OUT), jnp.float32),
            jax.ShapeDtypeStruct((N_PAD,), jnp.float32),
            jax.ShapeDtypeStruct((N_PAD,), jnp.float32),
        ],
    )(z, deg_pair)


def _mid_body(tp_ref, dinv2_ref, s_ref):
    t = tp_ref[0] + tp_ref[1]
    s_ref[...] = t * dinv2_ref[...][:, None]


def _mid_call(t_pair, dinv2):
    bm = 2048
    return pl.pallas_call(
        _mid_body,
        grid=(N_PAD // bm,),
        in_specs=[
            pl.BlockSpec((2, bm, F_OUT), lambda i: (0, i, 0)),
            pl.BlockSpec((bm,), lambda i: (i,)),
        ],
        out_specs=pl.BlockSpec((bm, F_OUT), lambda i: (i, 0)),
        out_shape=jax.ShapeDtypeStruct((N_PAD, F_OUT), jnp.float32),
    )(t_pair, dinv2)


def _fin_body(tp_ref, dinv_ref, b_ref, o_ref):
    t = tp_ref[0] + tp_ref[1]
    o_ref[...] = t * dinv_ref[...][:, None] + b_ref[...][None, :]


def _fin_call(t_pair, dinv, b):
    bm = 2048
    return pl.pallas_call(
        _fin_body,
        grid=(N_PAD // bm,),
        in_specs=[
            pl.BlockSpec((2, bm, F_OUT), lambda i: (0, i, 0)),
            pl.BlockSpec((bm,), lambda i: (i,)),
            pl.BlockSpec((F_OUT,), lambda i: (0,)),
        ],
        out_specs=pl.BlockSpec((bm, F_OUT), lambda i: (i, 0)),
        out_shape=jax.ShapeDtypeStruct((N, F_OUT), jnp.float32),
    )(t_pair, dinv, b)


def kernel(x, adj, W, b):
    adj2 = adj.astype(jnp.int32)

    deg_pair = _deg_kernel(adj2)
    z = _zmm_call(x, W)
    s0, dinv, dinv2 = _scale_call(z, deg_pair)
    t0 = _prop_kernel(s0, adj2)
    s1 = _mid_call(t0, dinv2)
    t1 = _prop_kernel(s1, adj2)
    return _fin_call(t1, dinv, b)
